# baseline (speedup 1.0000x reference)
import functools

import jax
import jax.numpy as jnp
from jax import lax
from jax.experimental import pallas as pl
from jax.experimental.pallas import tpu as pltpu
from jax.experimental.pallas import tpu_sc as plsc

L = 16


def _ceil16(n):
    return (n + L - 1) // L * L


def _pad_idx(idx, n_pad):
    n = idx.shape[0]
    idx = idx.astype(jnp.int32)
    if n_pad == n:
        return idx
    return jnp.concatenate([idx, jnp.broadcast_to(idx[-1], (n_pad - n,))])


@functools.partial(jax.jit, static_argnames=("n_rows", "n_feat", "sizes"))
def _router(x_flat, idx0, idx1, idx2, *, n_rows, n_feat, sizes):
    NW = 32
    CH = 8
    rows_per_w = n_rows // NW
    n_chunks = rows_per_w // CH
    chunk_elems = CH * n_feat

    n_pads = tuple(_ceil16(s) for s in sizes)
    out_type = tuple(
        jax.ShapeDtypeStruct((n_rows * s,), jnp.float32) for s in sizes)

    mesh = plsc.VectorSubcoreMesh(core_axis_name="c", subcore_axis_name="s")

    scratch = [
        pltpu.VMEM((chunk_elems,), jnp.float32),
        pltpu.VMEM((CH * sizes[0] + L,), jnp.float32),
        pltpu.VMEM((CH * sizes[1] + L,), jnp.float32),
        pltpu.VMEM((CH * sizes[2] + L,), jnp.float32),
        pltpu.VMEM((n_pads[0],), jnp.int32),
        pltpu.VMEM((n_pads[1],), jnp.int32),
        pltpu.VMEM((n_pads[2],), jnp.int32),
    ]

    @functools.partial(
        pl.kernel, out_type=out_type, mesh=mesh, scratch_types=scratch,
        compiler_params=pltpu.CompilerParams(needs_layout_passes=False))
    def k(x_hbm, i0_hbm, i1_hbm, i2_hbm, o0_hbm, o1_hbm, o2_hbm,
          inbuf, ob0, ob1, ob2, iv0, iv1, iv2):
        wid = lax.axis_index("s") * 2 + lax.axis_index("c")
        pltpu.sync_copy(i0_hbm, iv0)
        pltpu.sync_copy(i1_hbm, iv1)
        pltpu.sync_copy(i2_hbm, iv2)

        lanes = lax.iota(jnp.int32, L)
        obufs = (ob0, ob1, ob2)
        ivs = (iv0, iv1, iv2)
        outs = (o0_hbm, o1_hbm, o2_hbm)
        row0 = wid * rows_per_w

        @pl.loop(0, n_chunks)
        def _chunk(c):
            r0 = row0 + c * CH
            pltpu.sync_copy(x_hbm.at[pl.ds(r0 * n_feat, chunk_elems)], inbuf)
            for b in range(3):
                nb = sizes[b]
                nfull = nb // L
                rem = nb - nfull * L
                iv, ob = ivs[b], obufs[b]

                @pl.loop(0, nfull)
                def _vec(j):
                    idxv = iv[pl.ds(j * L, L)]
                    for r in range(CH):
                        v = plsc.load_gather(inbuf, [idxv + r * n_feat])
                        ob[pl.ds(r * nb + j * L, L)] = v

                if rem:
                    idxv = iv[pl.ds(nfull * L, L)]
                    mask = lanes < rem
                    for r in range(CH):
                        v = plsc.load_gather(inbuf, [idxv + r * n_feat])
                        plsc.store_scatter(
                            ob, [lanes + (r * nb + nfull * L)], v, mask=mask)
                pltpu.sync_copy(ob.at[pl.ds(0, CH * nb)],
                                outs[b].at[pl.ds(r0 * nb, CH * nb)])

    return k(x_flat,
             _pad_idx(idx0, n_pads[0]),
             _pad_idx(idx1, n_pads[1]),
             _pad_idx(idx2, n_pads[2]))


def kernel(x, void_dims, identity_dims, prime_dims):
    B, T, F = x.shape
    n_rows = B * T
    sizes = (void_dims.shape[0], identity_dims.shape[0], prime_dims.shape[0])
    o0, o1, o2 = _router(
        x.reshape(n_rows * F), void_dims, identity_dims, prime_dims,
        n_rows=n_rows, n_feat=F, sizes=sizes)
    return (o0.reshape(B, T, sizes[0]),
            o1.reshape(B, T, sizes[1]),
            o2.reshape(B, T, sizes[2]))


# speedup vs baseline: 1.1753x; 1.1753x over previous
import functools

import jax
import jax.numpy as jnp
from jax import lax
from jax.experimental import pallas as pl
from jax.experimental.pallas import tpu as pltpu
from jax.experimental.pallas import tpu_sc as plsc

L = 16


def _ceil16(n):
    return (n + L - 1) // L * L


def _pad_idx(idx, n_pad):
    n = idx.shape[0]
    idx = idx.astype(jnp.int32)
    if n_pad == n:
        return idx
    return jnp.concatenate([idx, jnp.broadcast_to(idx[-1], (n_pad - n,))])


@functools.partial(jax.jit, static_argnames=("n_rows", "n_feat", "sizes"))
def _router(x_flat, idx0, idx1, idx2, *, n_rows, n_feat, sizes):
    NW = 32
    CH = 8
    rows_per_w = n_rows // NW
    n_chunks = rows_per_w // CH
    assert n_chunks % 2 == 0
    chunk_elems = CH * n_feat

    n_pads = tuple(_ceil16(s) for s in sizes)
    out_type = tuple(
        jax.ShapeDtypeStruct((n_rows * s,), jnp.float32) for s in sizes)

    ostride = tuple(-(-(CH * s + L) // 128) * 128 for s in sizes)

    mesh = plsc.VectorSubcoreMesh(core_axis_name="c", subcore_axis_name="s")

    scratch = [
        pltpu.VMEM((2 * chunk_elems,), jnp.float32),
        pltpu.VMEM((2 * ostride[0],), jnp.float32),
        pltpu.VMEM((2 * ostride[1],), jnp.float32),
        pltpu.VMEM((2 * ostride[2],), jnp.float32),
        pltpu.VMEM((n_pads[0],), jnp.int32),
        pltpu.VMEM((n_pads[1],), jnp.int32),
        pltpu.VMEM((n_pads[2],), jnp.int32),
        pltpu.SemaphoreType.DMA,
        pltpu.SemaphoreType.DMA,
        pltpu.SemaphoreType.DMA,
        pltpu.SemaphoreType.DMA,
    ]

    @functools.partial(
        pl.kernel, out_type=out_type, mesh=mesh, scratch_types=scratch,
        compiler_params=pltpu.CompilerParams(needs_layout_passes=False))
    def k(x_hbm, i0_hbm, i1_hbm, i2_hbm, o0_hbm, o1_hbm, o2_hbm,
          inbuf, ob0, ob1, ob2, iv0, iv1, iv2,
          in_sem0, in_sem1, out_sem0, out_sem1):
        wid = lax.axis_index("s") * 2 + lax.axis_index("c")
        pltpu.sync_copy(i0_hbm, iv0)
        pltpu.sync_copy(i1_hbm, iv1)
        pltpu.sync_copy(i2_hbm, iv2)

        lanes = lax.iota(jnp.int32, L)
        obufs = (ob0, ob1, ob2)
        ivs = (iv0, iv1, iv2)
        outs = (o0_hbm, o1_hbm, o2_hbm)
        in_sems = (in_sem0, in_sem1)
        out_sems = (out_sem0, out_sem1)
        row0 = wid * rows_per_w

        def in_copy(c, s):
            return pltpu.make_async_copy(
                x_hbm.at[pl.ds((row0 + c * CH) * n_feat, chunk_elems)],
                inbuf.at[pl.ds(s * chunk_elems, chunk_elems)], in_sems[s])

        def out_copy(c, s, b):
            nb = sizes[b]
            return pltpu.make_async_copy(
                obufs[b].at[pl.ds(s * ostride[b], CH * nb)],
                outs[b].at[pl.ds((row0 + c * CH) * nb, CH * nb)],
                out_sems[s])

        def gather_chunk(s):
            src = inbuf.at[pl.ds(s * chunk_elems, chunk_elems)]
            for b in range(3):
                nb = sizes[b]
                nfull = nb // L
                rem = nb - nfull * L
                iv = ivs[b]
                ob = obufs[b].at[pl.ds(s * ostride[b], ostride[b])]

                @pl.loop(0, nfull, unroll=4)
                def _vec(j):
                    idxv = iv[pl.ds(j * L, L)]
                    for r in range(CH):
                        v = plsc.load_gather(src, [idxv + r * n_feat])
                        ob[pl.ds(r * nb + j * L, L)] = v

                idxv = iv[pl.ds(nfull * L, L)]
                mask = lanes < rem
                for r in range(CH):
                    v = plsc.load_gather(src, [idxv + r * n_feat])
                    plsc.store_scatter(
                        ob, [lanes + (r * nb + nfull * L)], v, mask=mask)

        in_copy(0, 0).start()
        in_copy(1, 1).start()

        @pl.loop(0, n_chunks, step=2)
        def _chunk(c):
            for s in range(2):
                kk = c + s
                in_copy(kk, s).wait()

                @pl.when(kk >= 2)
                def _drain():
                    for b in range(3):
                        out_copy(kk, s, b).wait()

                gather_chunk(s)
                for b in range(3):
                    out_copy(kk, s, b).start()

                @pl.when(kk + 2 < n_chunks)
                def _next():
                    in_copy(kk + 2, s).start()

        for s in range(2):
            for b in range(3):
                out_copy(n_chunks - 2 + s, s, b).wait()

    return k(x_flat,
             _pad_idx(idx0, n_pads[0]),
             _pad_idx(idx1, n_pads[1]),
             _pad_idx(idx2, n_pads[2]))


def kernel(x, void_dims, identity_dims, prime_dims):
    B, T, F = x.shape
    n_rows = B * T
    sizes = (void_dims.shape[0], identity_dims.shape[0], prime_dims.shape[0])
    o0, o1, o2 = _router(
        x.reshape(n_rows * F), void_dims, identity_dims, prime_dims,
        n_rows=n_rows, n_feat=F, sizes=sizes)
    return (o0.reshape(B, T, sizes[0]),
            o1.reshape(B, T, sizes[1]),
            o2.reshape(B, T, sizes[2]))


# speedup vs baseline: 1.6723x; 1.4229x over previous
import functools

import jax
import jax.numpy as jnp
from jax import lax
from jax.experimental import pallas as pl
from jax.experimental.pallas import tpu as pltpu
from jax.experimental.pallas import tpu_sc as plsc

L = 16


def _ceil16(n):
    return (n + L - 1) // L * L


def _pad_idx(idx, n_pad):
    n = idx.shape[0]
    idx = idx.astype(jnp.int32)
    if n_pad == n:
        return idx
    return jnp.concatenate([idx, jnp.broadcast_to(idx[-1], (n_pad - n,))])


@functools.partial(jax.jit, static_argnames=("n_rows", "n_feat", "sizes"))
def _router(x, idx0, idx1, idx2, *, n_rows, n_feat, sizes):
    NW = 32
    CH = 8
    rows_per_w = n_rows // NW
    n_chunks = rows_per_w // CH
    assert n_chunks % 2 == 0

    n_pads = tuple(_ceil16(s) for s in sizes)
    out_type = tuple(
        jax.ShapeDtypeStruct((n_rows, s), jnp.float32) for s in sizes)

    mesh = plsc.VectorSubcoreMesh(core_axis_name="c", subcore_axis_name="s")

    scratch = [
        pltpu.VMEM((2 * CH, n_feat), jnp.float32),
        pltpu.VMEM((2 * CH, sizes[0]), jnp.float32),
        pltpu.VMEM((2 * CH, sizes[1]), jnp.float32),
        pltpu.VMEM((2 * CH, sizes[2]), jnp.float32),
        pltpu.VMEM((n_pads[0],), jnp.int32),
        pltpu.VMEM((n_pads[1],), jnp.int32),
        pltpu.VMEM((n_pads[2],), jnp.int32),
        pltpu.SemaphoreType.DMA,
        pltpu.SemaphoreType.DMA,
        pltpu.SemaphoreType.DMA,
        pltpu.SemaphoreType.DMA,
    ]

    @functools.partial(
        pl.kernel, out_type=out_type, mesh=mesh, scratch_types=scratch,
        compiler_params=pltpu.CompilerParams(
            needs_layout_passes=False, use_tc_tiling_on_sc=True))
    def k(x_hbm, i0_hbm, i1_hbm, i2_hbm, o0_hbm, o1_hbm, o2_hbm,
          inbuf, ob0, ob1, ob2, iv0, iv1, iv2,
          in_sem0, in_sem1, out_sem0, out_sem1):
        wid = lax.axis_index("s") * 2 + lax.axis_index("c")
        pltpu.sync_copy(i0_hbm, iv0)
        pltpu.sync_copy(i1_hbm, iv1)
        pltpu.sync_copy(i2_hbm, iv2)

        lanes = lax.iota(jnp.int32, L)
        obufs = (ob0, ob1, ob2)
        ivs = (iv0, iv1, iv2)
        outs = (o0_hbm, o1_hbm, o2_hbm)
        in_sems = (in_sem0, in_sem1)
        out_sems = (out_sem0, out_sem1)
        row0 = wid * rows_per_w

        def in_copy(c, s):
            return pltpu.make_async_copy(
                x_hbm.at[pl.ds(row0 + c * CH, CH)],
                inbuf.at[pl.ds(s * CH, CH)],
                in_sems[s])

        def out_copy(c, s, b):
            return pltpu.make_async_copy(
                obufs[b].at[pl.ds(s * CH, CH)],
                outs[b].at[pl.ds(row0 + c * CH, CH)],
                out_sems[s])

        def gather_chunk(s):
            for b in range(3):
                nb = sizes[b]
                nfull = nb // L
                rem = nb - nfull * L
                iv, ob = ivs[b], obufs[b]

                @pl.loop(0, nfull, unroll=4)
                def _vec(j):
                    idxv = iv[pl.ds(j * L, L)]
                    ocol = lanes + j * L
                    for r in range(CH):
                        rv = jnp.full((L,), s * CH + r, jnp.int32)
                        v = plsc.load_gather(inbuf, [rv, idxv])
                        plsc.store_scatter(ob, [rv, ocol], v)

                idxv = iv[pl.ds(nfull * L, L)]
                ocol = lanes + nfull * L
                mask = lanes < rem
                for r in range(CH):
                    rv = jnp.full((L,), s * CH + r, jnp.int32)
                    v = plsc.load_gather(inbuf, [rv, idxv])
                    plsc.store_scatter(ob, [rv, ocol], v, mask=mask)

        in_copy(0, 0).start()
        in_copy(1, 1).start()

        @pl.loop(0, n_chunks, step=2)
        def _chunk(c):
            for s in range(2):
                kk = c + s
                in_copy(kk, s).wait()

                @pl.when(kk >= 2)
                def _drain():
                    for b in range(3):
                        out_copy(kk, s, b).wait()

                gather_chunk(s)
                for b in range(3):
                    out_copy(kk, s, b).start()

                @pl.when(kk + 2 < n_chunks)
                def _next():
                    in_copy(kk + 2, s).start()

        for s in range(2):
            for b in range(3):
                out_copy(n_chunks - 2 + s, s, b).wait()

    return k(x,
             _pad_idx(idx0, n_pads[0]),
             _pad_idx(idx1, n_pads[1]),
             _pad_idx(idx2, n_pads[2]))


def kernel(x, void_dims, identity_dims, prime_dims):
    B, T, F = x.shape
    n_rows = B * T
    sizes = (void_dims.shape[0], identity_dims.shape[0], prime_dims.shape[0])
    o0, o1, o2 = _router(
        x.reshape(n_rows, F), void_dims, identity_dims, prime_dims,
        n_rows=n_rows, n_feat=F, sizes=sizes)
    return (o0.reshape(B, T, sizes[0]),
            o1.reshape(B, T, sizes[1]),
            o2.reshape(B, T, sizes[2]))
